# baseline (speedup 1.0000x reference)
import functools

import jax
import jax.numpy as jnp
from jax import lax
from jax.experimental import pallas as pl
from jax.experimental.pallas import tpu as pltpu
from jax.experimental.pallas import tpu_sc as plsc

H = 2048
E = 8
F = 512
TOPK = 2
NGROUP = 4
GSZ = E // NGROUP
TOPKG = 2
SF = 1024
T = 2048

BLK = 128
NB = 40
NS = NB * BLK

TC = 16
CT = T // TC

NTILE = 32
TPW = T // NTILE
NG = TPW // 16


def _router_body(x_ref, gw_ref, metat_ref, w01_ref, offr_ref, be_ref,
                 base_ref):
    c = pl.program_id(0)

    @pl.when(c == 0)
    def _():
        base_ref[...] = jnp.zeros_like(base_ref)

    x = x_ref[...]
    gw = gw_ref[...]
    logits = lax.dot_general(
        x, gw, (((1,), (1,)), ((), ())), preferred_element_type=jnp.float32)
    m = jnp.max(logits, axis=-1, keepdims=True)
    ex = jnp.exp(logits - m)
    scores = ex / jnp.sum(ex, axis=-1, keepdims=True)

    cols = [scores[:, i:i + 1] for i in range(E)]
    gexp = [jnp.maximum(cols[2 * g], cols[2 * g + 1]) for g in range(NGROUP)]
    grank = []
    for g in range(NGROUP):
        r = jnp.zeros_like(gexp[0])
        for j in range(NGROUP):
            if j == g:
                continue
            beat = (gexp[j] > gexp[g]) | ((gexp[j] == gexp[g]) & (j < g))
            r = r + jnp.where(beat, 1.0, 0.0)
        grank.append(r)
    mcols = [jnp.where(grank[e // GSZ] < float(TOPKG), cols[e], 0.0)
             for e in range(E)]
    rank8 = []
    for e in range(E):
        r = jnp.zeros_like(mcols[0])
        for j in range(E):
            if j == e:
                continue
            beat = (mcols[j] > mcols[e]) | ((mcols[j] == mcols[e]) & (j < e))
            r = r + jnp.where(beat, 1.0, 0.0)
        rank8.append(r)

    sel0 = [rank8[e] == 0.0 for e in range(E)]
    sel1 = [rank8[e] == 1.0 for e in range(E)]
    zero = jnp.zeros_like(cols[0])
    e0 = sum(jnp.where(sel0[e], float(e), zero) for e in range(E))
    e1 = sum(jnp.where(sel1[e], float(e), zero) for e in range(E))
    w0 = sum(jnp.where(sel0[e], mcols[e], zero) for e in range(E))
    w1 = sum(jnp.where(sel1[e], mcols[e], zero) for e in range(E))

    oh = jnp.concatenate(
        [jnp.where(sel0[e] | sel1[e], 1.0, 0.0) for e in range(E)], axis=1)
    ii = lax.broadcasted_iota(jnp.int32, (CT, CT), 0)
    jj = lax.broadcasted_iota(jnp.int32, (CT, CT), 1)
    lexc = jnp.where(ii > jj, 1.0, 0.0)
    within = lax.dot_general(
        lexc, oh, (((1,), (0,)), ((), ())),
        preferred_element_type=jnp.float32)
    rank_te = within + base_ref[...]
    r0 = sum(jnp.where(sel0[e], rank_te[:, e:e + 1], zero) for e in range(E))
    r1 = sum(jnp.where(sel1[e], rank_te[:, e:e + 1], zero) for e in range(E))

    m8 = jnp.concatenate([e0, e1, r0, r1, w0, w1, zero, zero], axis=1)
    ident = jnp.where(ii == jj, 1.0, 0.0)
    metat_ref[...] = lax.dot_general(
        m8, ident, (((0,), (0,)), ((), ())),
        preferred_element_type=jnp.float32,
        precision=lax.Precision.HIGHEST)

    w01_ref[...] = jnp.concatenate(
        [jnp.broadcast_to(w0, (CT, 16)), jnp.broadcast_to(w1, (CT, 16))],
        axis=1)

    base_ref[...] += jnp.sum(oh, axis=0, keepdims=True)

    @pl.when(c == TC - 1)
    def _():
        cnt = base_ref[...]
        padblk = jnp.floor((cnt + float(BLK - 1)) * (1.0 / BLK))
        offs = []
        run = jnp.zeros_like(padblk[:, 0:1])
        ends = []
        for e in range(E):
            offs.append(run * float(BLK))
            run = run + padblk[:, e:e + 1]
            ends.append(run)
        off2 = jnp.concatenate(offs, axis=1)
        i8a = lax.broadcasted_iota(jnp.int32, (E, E), 0)
        i8b = lax.broadcasted_iota(jnp.int32, (E, E), 1)
        ident8 = jnp.where(i8a == i8b, 1.0, 0.0)
        offcol = lax.dot_general(
            ident8, off2, (((1,), (1,)), ((), ())),
            preferred_element_type=jnp.float32,
            precision=lax.Precision.HIGHEST)
        offr_ref[...] = lax.dot_general(
            offcol, jnp.ones((1, 16), jnp.float32),
            (((1,), (0,)), ((), ())),
            preferred_element_type=jnp.float32,
            precision=lax.Precision.HIGHEST)
        bi = lax.broadcasted_iota(jnp.int32, (1, NB), 1).astype(jnp.float32)
        be = sum(jnp.where(bi >= ends[e], 1.0, 0.0) for e in range(E))
        be = jnp.minimum(be, float(E - 1))
        be_ref[...] = be.astype(jnp.int32).reshape(1, 1, NB)


def _router(x, gate_w):
    return pl.pallas_call(
        _router_body,
        grid=(TC,),
        in_specs=[
            pl.BlockSpec((CT, H), lambda c: (c, 0)),
            pl.BlockSpec((E, H), lambda c: (0, 0)),
        ],
        out_specs=(
            pl.BlockSpec((E, CT), lambda c: (0, c)),
            pl.BlockSpec((CT, 32), lambda c: (c, 0)),
            pl.BlockSpec((E, 16), lambda c: (0, 0)),
            pl.BlockSpec((1, 1, NB), lambda c: (0, 0, 0)),
        ),
        out_shape=(
            jax.ShapeDtypeStruct((E, T), jnp.float32),
            jax.ShapeDtypeStruct((T, 32), jnp.float32),
            jax.ShapeDtypeStruct((E, 16), jnp.float32),
            jax.ShapeDtypeStruct((1, 1, NB), jnp.int32),
        ),
        scratch_shapes=[pltpu.VMEM((1, E), jnp.float32)],
    )(x, gate_w)


def _dispatch_body(x_hbm, metat_hbm, offr_hbm, xs_hbm, pos_hbm,
                   mt_v, off_v, pos_v, rows_v, sem):
    wid = lax.axis_index("s") * 2 + lax.axis_index("c")
    base = wid * TPW
    for r in range(4):
        pltpu.sync_copy(metat_hbm.at[r, pl.ds(base, TPW)],
                        mt_v.at[pl.ds(r * TPW, TPW)])
    pltpu.sync_copy(offr_hbm, off_v)
    posvecs = []
    for g in range(NG):
        ev0 = mt_v[pl.ds(0 * TPW + g * 16, 16)]
        ev1 = mt_v[pl.ds(1 * TPW + g * 16, 16)]
        p0 = mt_v[pl.ds(2 * TPW + g * 16, 16)]
        p1 = mt_v[pl.ds(3 * TPW + g * 16, 16)]
        for e in range(E):
            ov = off_v[pl.ds(e * 16, 16)]
            fe = float(e)
            p0 = p0 + jnp.where(ev0 == fe, ov, 0.0)
            p1 = p1 + jnp.where(ev1 == fe, ov, 0.0)
        i0 = p0.astype(jnp.int32)
        i1 = p1.astype(jnp.int32)
        pos_v[pl.ds(g * 16, 16)] = i0
        pos_v[pl.ds(TPW + g * 16, 16)] = i1
        posvecs.append((i0, i1))
    pltpu.sync_copy(pos_v, pos_hbm.at[pl.ds(wid * 2 * TPW, 2 * TPW)])
    for g in range(NG):
        i0, i1 = posvecs[g]
        pltpu.sync_copy(x_hbm.at[pl.ds(base + g * 16, 16)], rows_v)
        c0 = pltpu.async_copy(rows_v, xs_hbm.at[i0], sem)
        c1 = pltpu.async_copy(rows_v, xs_hbm.at[i1], sem)
        c0.wait()
        c1.wait()


def _dispatch(x, metat, offr_flat):
    mesh = plsc.VectorSubcoreMesh(core_axis_name="c", subcore_axis_name="s")
    return pl.kernel(
        _dispatch_body,
        mesh=mesh,
        out_type=(
            jax.ShapeDtypeStruct((NS, H), jnp.float32),
            jax.ShapeDtypeStruct((2 * T,), jnp.int32),
        ),
        scratch_types=[
            pltpu.VMEM((4 * TPW,), jnp.float32),
            pltpu.VMEM((E * 16,), jnp.float32),
            pltpu.VMEM((2 * TPW,), jnp.int32),
            pltpu.VMEM((16, H), jnp.float32),
            pltpu.SemaphoreType.DMA,
        ],
    )(x, metat, offr_flat)


def _mlp_body(be_ref, xs_ref, wg_ref, wu_ref, wd_ref, ys_ref):
    xb = xs_ref[...].astype(jnp.bfloat16)
    wg = wg_ref[0].astype(jnp.bfloat16)
    wu = wu_ref[0].astype(jnp.bfloat16)
    wd = wd_ref[0].astype(jnp.bfloat16)
    g = lax.dot_general(xb, wg, (((1,), (1,)), ((), ())),
                        preferred_element_type=jnp.float32)
    u = lax.dot_general(xb, wu, (((1,), (1,)), ((), ())),
                        preferred_element_type=jnp.float32)
    h = (g * jax.nn.sigmoid(g) * u).astype(jnp.bfloat16)
    ys_ref[...] = lax.dot_general(h, wd, (((1,), (1,)), ((), ())),
                                  preferred_element_type=jnp.float32)


def _mlp(be, xs, gate_ws, up_ws, down_ws):
    return pl.pallas_call(
        _mlp_body,
        grid_spec=pltpu.PrefetchScalarGridSpec(
            num_scalar_prefetch=1,
            grid=(NB,),
            in_specs=[
                pl.BlockSpec((BLK, H), lambda b, be: (b, 0)),
                pl.BlockSpec((1, F, H), lambda b, be: (be[b], 0, 0)),
                pl.BlockSpec((1, F, H), lambda b, be: (be[b], 0, 0)),
                pl.BlockSpec((1, H, F), lambda b, be: (be[b], 0, 0)),
            ],
            out_specs=pl.BlockSpec((BLK, H), lambda b, be: (b, 0)),
        ),
        out_shape=jax.ShapeDtypeStruct((NS, H), jnp.float32),
    )(be, xs, gate_ws, up_ws, down_ws)


SFC = 4
SFB = SF // SFC


def _shared_body(xbf_ref, wg_ref, wu_ref, wd_ref, out_ref):
    step = pl.program_id(0)

    @pl.when(step == 0)
    def _():
        out_ref[...] = jnp.zeros_like(out_ref)

    xb = xbf_ref[...].astype(jnp.bfloat16)
    wg = wg_ref[...].astype(jnp.bfloat16)
    wu = wu_ref[...].astype(jnp.bfloat16)
    wd = wd_ref[...].astype(jnp.bfloat16)
    g = lax.dot_general(xb, wg, (((1,), (1,)), ((), ())),
                        preferred_element_type=jnp.float32)
    u = lax.dot_general(xb, wu, (((1,), (1,)), ((), ())),
                        preferred_element_type=jnp.float32)
    h = (g * jax.nn.sigmoid(g) * u).astype(jnp.bfloat16)
    out_ref[...] += lax.dot_general(h, wd, (((1,), (1,)), ((), ())),
                                    preferred_element_type=jnp.float32)


def _shared(x, sg_w, su_w, sd_w):
    return pl.pallas_call(
        _shared_body,
        grid=(SFC,),
        in_specs=[
            pl.BlockSpec((T, H), lambda i: (0, 0)),
            pl.BlockSpec((SFB, H), lambda i: (i, 0)),
            pl.BlockSpec((SFB, H), lambda i: (i, 0)),
            pl.BlockSpec((H, SFB), lambda i: (0, i)),
        ],
        out_specs=pl.BlockSpec((T, H), lambda i: (0, 0)),
        out_shape=jax.ShapeDtypeStruct((T, H), jnp.float32),
    )(x, sg_w, su_w, sd_w)


def _combine_body(ys_hbm, shared_hbm, w01_hbm, pos_hbm, out_hbm,
                  w_v, pos_v, buf0, buf1, sbuf, sem):
    wid = lax.axis_index("s") * 2 + lax.axis_index("c")
    base = wid * TPW
    pltpu.sync_copy(w01_hbm.at[pl.ds(base * 32, TPW * 32)], w_v)
    pltpu.sync_copy(pos_hbm.at[pl.ds(wid * 2 * TPW, 2 * TPW)], pos_v)
    for g in range(NG):
        i0 = pos_v[pl.ds(g * 16, 16)]
        i1 = pos_v[pl.ds(TPW + g * 16, 16)]
        c0 = pltpu.async_copy(ys_hbm.at[i0], buf0, sem)
        c1 = pltpu.async_copy(ys_hbm.at[i1], buf1, sem)
        pltpu.sync_copy(shared_hbm.at[pl.ds(base + g * 16, 16)], sbuf)
        c0.wait()
        c1.wait()
        for i in range(16):
            tok = g * 16 + i
            wv0 = w_v[pl.ds(tok * 32, 16)]
            wv1 = w_v[pl.ds(tok * 32 + 16, 16)]

            def seg(s, _, i=i, wv0=wv0, wv1=wv1):
                sl = pl.ds(s * 16, 16)
                buf0[i, sl] = (wv0 * buf0[i, sl] + wv1 * buf1[i, sl]
                               + sbuf[i, sl])
                return 0

            lax.fori_loop(0, H // 16, seg, 0)
        pltpu.sync_copy(buf0, out_hbm.at[pl.ds(base + g * 16, 16)])


def _combine(ys, shared, w01_flat, pos):
    mesh = plsc.VectorSubcoreMesh(core_axis_name="c", subcore_axis_name="s")
    return pl.kernel(
        _combine_body,
        mesh=mesh,
        out_type=jax.ShapeDtypeStruct((T, H), jnp.float32),
        scratch_types=[
            pltpu.VMEM((TPW * 32,), jnp.float32),
            pltpu.VMEM((2 * TPW,), jnp.int32),
            pltpu.VMEM((16, H), jnp.float32),
            pltpu.VMEM((16, H), jnp.float32),
            pltpu.VMEM((16, H), jnp.float32),
            pltpu.SemaphoreType.DMA,
        ],
    )(ys, shared, w01_flat, pos)


def kernel(hidden_states, gate_w, gate_ws, up_ws, down_ws,
           shared_gate_w, shared_up_w, shared_down_w):
    metat, w01, offr, be3 = _router(hidden_states, gate_w)
    w01_flat = w01.reshape(T * 32)
    offr_flat = offr.reshape(E * 16)
    be = be3.reshape(NB)
    xs, pos = _dispatch(hidden_states, metat, offr_flat)
    ys = _mlp(be, xs, gate_ws, up_ws, down_ws)
    shared = _shared(hidden_states, shared_gate_w, shared_up_w,
                     shared_down_w)
    return _combine(ys, shared, w01_flat, pos)


# speedup vs baseline: 1.1578x; 1.1578x over previous
import functools

import jax
import jax.numpy as jnp
from jax import lax
from jax.experimental import pallas as pl
from jax.experimental.pallas import tpu as pltpu
from jax.experimental.pallas import tpu_sc as plsc

H = 2048
E = 8
F = 512
TOPK = 2
NGROUP = 4
GSZ = E // NGROUP
TOPKG = 2
SF = 1024
T = 2048

BLK = 128
NB = 40
NS = NB * BLK

TC = 16
CT = T // TC

NTILE = 32
TPW = T // NTILE
NG = TPW // 16


def _router_body(x_ref, gw_ref, metat_ref, w01_ref, offr_ref, be_ref,
                 xbf_ref, base_ref):
    c = pl.program_id(0)

    @pl.when(c == 0)
    def _():
        base_ref[...] = jnp.zeros_like(base_ref)

    x = x_ref[...]
    xbf_ref[...] = x.astype(jnp.bfloat16)
    gw = gw_ref[...]
    logits = lax.dot_general(
        x, gw, (((1,), (1,)), ((), ())), preferred_element_type=jnp.float32)
    m = jnp.max(logits, axis=-1, keepdims=True)
    ex = jnp.exp(logits - m)
    scores = ex / jnp.sum(ex, axis=-1, keepdims=True)

    cols = [scores[:, i:i + 1] for i in range(E)]
    gexp = [jnp.maximum(cols[2 * g], cols[2 * g + 1]) for g in range(NGROUP)]
    grank = []
    for g in range(NGROUP):
        r = jnp.zeros_like(gexp[0])
        for j in range(NGROUP):
            if j == g:
                continue
            beat = (gexp[j] > gexp[g]) | ((gexp[j] == gexp[g]) & (j < g))
            r = r + jnp.where(beat, 1.0, 0.0)
        grank.append(r)
    mcols = [jnp.where(grank[e // GSZ] < float(TOPKG), cols[e], 0.0)
             for e in range(E)]
    rank8 = []
    for e in range(E):
        r = jnp.zeros_like(mcols[0])
        for j in range(E):
            if j == e:
                continue
            beat = (mcols[j] > mcols[e]) | ((mcols[j] == mcols[e]) & (j < e))
            r = r + jnp.where(beat, 1.0, 0.0)
        rank8.append(r)

    sel0 = [rank8[e] == 0.0 for e in range(E)]
    sel1 = [rank8[e] == 1.0 for e in range(E)]
    zero = jnp.zeros_like(cols[0])
    e0 = sum(jnp.where(sel0[e], float(e), zero) for e in range(E))
    e1 = sum(jnp.where(sel1[e], float(e), zero) for e in range(E))
    w0 = sum(jnp.where(sel0[e], mcols[e], zero) for e in range(E))
    w1 = sum(jnp.where(sel1[e], mcols[e], zero) for e in range(E))

    oh = jnp.concatenate(
        [jnp.where(sel0[e] | sel1[e], 1.0, 0.0) for e in range(E)], axis=1)
    ii = lax.broadcasted_iota(jnp.int32, (CT, CT), 0)
    jj = lax.broadcasted_iota(jnp.int32, (CT, CT), 1)
    lexc = jnp.where(ii > jj, 1.0, 0.0)
    within = lax.dot_general(
        lexc, oh, (((1,), (0,)), ((), ())),
        preferred_element_type=jnp.float32)
    rank_te = within + base_ref[...]
    r0 = sum(jnp.where(sel0[e], rank_te[:, e:e + 1], zero) for e in range(E))
    r1 = sum(jnp.where(sel1[e], rank_te[:, e:e + 1], zero) for e in range(E))

    m8 = jnp.concatenate([e0, e1, r0, r1, w0, w1, zero, zero], axis=1)
    ident = jnp.where(ii == jj, 1.0, 0.0)
    metat_ref[...] = lax.dot_general(
        m8, ident, (((0,), (0,)), ((), ())),
        preferred_element_type=jnp.float32,
        precision=lax.Precision.HIGHEST)

    w01_ref[...] = jnp.concatenate(
        [jnp.broadcast_to(w0, (CT, 16)), jnp.broadcast_to(w1, (CT, 16))],
        axis=1)

    base_ref[...] += jnp.sum(oh, axis=0, keepdims=True)

    @pl.when(c == TC - 1)
    def _():
        cnt = base_ref[...]
        padblk = jnp.floor((cnt + float(BLK - 1)) * (1.0 / BLK))
        offs = []
        run = jnp.zeros_like(padblk[:, 0:1])
        ends = []
        for e in range(E):
            offs.append(run * float(BLK))
            run = run + padblk[:, e:e + 1]
            ends.append(run)
        off2 = jnp.concatenate(offs, axis=1)
        i8a = lax.broadcasted_iota(jnp.int32, (E, E), 0)
        i8b = lax.broadcasted_iota(jnp.int32, (E, E), 1)
        ident8 = jnp.where(i8a == i8b, 1.0, 0.0)
        offcol = lax.dot_general(
            ident8, off2, (((1,), (1,)), ((), ())),
            preferred_element_type=jnp.float32,
            precision=lax.Precision.HIGHEST)
        offr_ref[...] = lax.dot_general(
            offcol, jnp.ones((1, 16), jnp.float32),
            (((1,), (0,)), ((), ())),
            preferred_element_type=jnp.float32,
            precision=lax.Precision.HIGHEST)
        bi = lax.broadcasted_iota(jnp.int32, (1, NB), 1).astype(jnp.float32)
        be = sum(jnp.where(bi >= ends[e], 1.0, 0.0) for e in range(E))
        be = jnp.minimum(be, float(E - 1))
        be_ref[...] = be.astype(jnp.int32).reshape(1, 1, NB)


def _router(x, gate_w):
    return pl.pallas_call(
        _router_body,
        grid=(TC,),
        in_specs=[
            pl.BlockSpec((CT, H), lambda c: (c, 0)),
            pl.BlockSpec((E, H), lambda c: (0, 0)),
        ],
        out_specs=(
            pl.BlockSpec((E, CT), lambda c: (0, c)),
            pl.BlockSpec((CT, 32), lambda c: (c, 0)),
            pl.BlockSpec((E, 16), lambda c: (0, 0)),
            pl.BlockSpec((1, 1, NB), lambda c: (0, 0, 0)),
            pl.BlockSpec((CT, H), lambda c: (c, 0)),
        ),
        out_shape=(
            jax.ShapeDtypeStruct((E, T), jnp.float32),
            jax.ShapeDtypeStruct((T, 32), jnp.float32),
            jax.ShapeDtypeStruct((E, 16), jnp.float32),
            jax.ShapeDtypeStruct((1, 1, NB), jnp.int32),
            jax.ShapeDtypeStruct((T, H), jnp.bfloat16),
        ),
        scratch_shapes=[pltpu.VMEM((1, E), jnp.float32)],
    )(x, gate_w)


def _dispatch_body(x_hbm, metat_hbm, offr_hbm, xs_hbm, pos_hbm,
                   mt_v, off_v, pos_v, rows_v, sem_l, sem_s):
    wid = lax.axis_index("s") * 2 + lax.axis_index("c")
    base = wid * TPW
    for r in range(4):
        pltpu.sync_copy(metat_hbm.at[r, pl.ds(base, TPW)],
                        mt_v.at[pl.ds(r * TPW, TPW)])
    pltpu.sync_copy(offr_hbm, off_v)
    posvecs = []
    for g in range(NG):
        ev0 = mt_v[pl.ds(0 * TPW + g * 16, 16)]
        ev1 = mt_v[pl.ds(1 * TPW + g * 16, 16)]
        p0 = mt_v[pl.ds(2 * TPW + g * 16, 16)]
        p1 = mt_v[pl.ds(3 * TPW + g * 16, 16)]
        for e in range(E):
            ov = off_v[pl.ds(e * 16, 16)]
            fe = float(e)
            p0 = p0 + jnp.where(ev0 == fe, ov, 0.0)
            p1 = p1 + jnp.where(ev1 == fe, ov, 0.0)
        i0 = p0.astype(jnp.int32)
        i1 = p1.astype(jnp.int32)
        pos_v[pl.ds(g * 16, 16)] = i0
        pos_v[pl.ds(TPW + g * 16, 16)] = i1
        posvecs.append((i0, i1))
    pltpu.sync_copy(pos_v, pos_hbm.at[pl.ds(wid * 2 * TPW, 2 * TPW)])
    loads = [None] * NG
    scts = [None] * NG

    def _load(g):
        return pltpu.async_copy(
            x_hbm.at[pl.ds(base + g * 16, 16)], rows_v.at[g % 2], sem_l)

    loads[0] = _load(0)
    for g in range(NG):
        loads[g].wait()
        if g >= 1:
            scts[g - 1][0].wait()
            scts[g - 1][1].wait()
        if g + 1 < NG:
            loads[g + 1] = _load(g + 1)
        i0, i1 = posvecs[g]
        scts[g] = (
            pltpu.async_copy(rows_v.at[g % 2], xs_hbm.at[i0], sem_s),
            pltpu.async_copy(rows_v.at[g % 2], xs_hbm.at[i1], sem_s),
        )
    scts[NG - 1][0].wait()
    scts[NG - 1][1].wait()


def _dispatch(x, metat, offr_flat):
    mesh = plsc.VectorSubcoreMesh(core_axis_name="c", subcore_axis_name="s")
    return pl.kernel(
        _dispatch_body,
        mesh=mesh,
        out_type=(
            jax.ShapeDtypeStruct((NS, H), jnp.float32),
            jax.ShapeDtypeStruct((2 * T,), jnp.int32),
        ),
        scratch_types=[
            pltpu.VMEM((4 * TPW,), jnp.float32),
            pltpu.VMEM((E * 16,), jnp.float32),
            pltpu.VMEM((2 * TPW,), jnp.int32),
            pltpu.VMEM((2, 16, H), jnp.float32),
            pltpu.SemaphoreType.DMA,
            pltpu.SemaphoreType.DMA,
        ],
    )(x, metat, offr_flat)


def _mlp_body(be_ref, xs_ref, wg_ref, wu_ref, wd_ref, ys_ref):
    xb = xs_ref[...].astype(jnp.bfloat16)
    wg = wg_ref[0].astype(jnp.bfloat16)
    wu = wu_ref[0].astype(jnp.bfloat16)
    wd = wd_ref[0].astype(jnp.bfloat16)
    g = lax.dot_general(xb, wg, (((1,), (1,)), ((), ())),
                        preferred_element_type=jnp.float32)
    u = lax.dot_general(xb, wu, (((1,), (1,)), ((), ())),
                        preferred_element_type=jnp.float32)
    h = (g * jax.nn.sigmoid(g) * u).astype(jnp.bfloat16)
    ys_ref[...] = lax.dot_general(h, wd, (((1,), (1,)), ((), ())),
                                  preferred_element_type=jnp.float32)


def _mlp(be, xs, gate_ws, up_ws, down_ws):
    return pl.pallas_call(
        _mlp_body,
        grid_spec=pltpu.PrefetchScalarGridSpec(
            num_scalar_prefetch=1,
            grid=(NB,),
            in_specs=[
                pl.BlockSpec((BLK, H), lambda b, be: (b, 0)),
                pl.BlockSpec((1, F, H), lambda b, be: (be[b], 0, 0)),
                pl.BlockSpec((1, F, H), lambda b, be: (be[b], 0, 0)),
                pl.BlockSpec((1, H, F), lambda b, be: (be[b], 0, 0)),
            ],
            out_specs=pl.BlockSpec((BLK, H), lambda b, be: (b, 0)),
        ),
        out_shape=jax.ShapeDtypeStruct((NS, H), jnp.float32),
    )(be, xs, gate_ws, up_ws, down_ws)


SBT = 8
SBR = T // SBT


def _shared_body(xbf_ref, wg_ref, wu_ref, wd_ref, y0_ref, y1_ref, w01_ref,
                 out_ref):
    xb = xbf_ref[...]
    wg = wg_ref[...].astype(jnp.bfloat16)
    wu = wu_ref[...].astype(jnp.bfloat16)
    wd = wd_ref[...].astype(jnp.bfloat16)
    g = lax.dot_general(xb, wg, (((1,), (1,)), ((), ())),
                        preferred_element_type=jnp.float32)
    u = lax.dot_general(xb, wu, (((1,), (1,)), ((), ())),
                        preferred_element_type=jnp.float32)
    h = (g * jax.nn.sigmoid(g) * u).astype(jnp.bfloat16)
    shared = lax.dot_general(h, wd, (((1,), (1,)), ((), ())),
                             preferred_element_type=jnp.float32)
    w0 = w01_ref[:, 0:1]
    w1 = w01_ref[:, 16:17]
    out_ref[...] = (shared
                    + w0 * y0_ref[...].astype(jnp.float32)
                    + w1 * y1_ref[...].astype(jnp.float32))


def _shared(xbf, sg_w, su_w, sd_w, y0, y1, w01):
    return pl.pallas_call(
        _shared_body,
        grid=(SBT,),
        in_specs=[
            pl.BlockSpec((SBR, H), lambda i: (i, 0)),
            pl.BlockSpec((SF, H), lambda i: (0, 0)),
            pl.BlockSpec((SF, H), lambda i: (0, 0)),
            pl.BlockSpec((H, SF), lambda i: (0, 0)),
            pl.BlockSpec((SBR, H), lambda i: (i, 0)),
            pl.BlockSpec((SBR, H), lambda i: (i, 0)),
            pl.BlockSpec((SBR, 32), lambda i: (i, 0)),
        ],
        out_specs=pl.BlockSpec((SBR, H), lambda i: (i, 0)),
        out_shape=jax.ShapeDtypeStruct((T, H), jnp.float32),
    )(xbf, sg_w, su_w, sd_w, y0, y1, w01)


def _combine_body(ys_hbm, pos_hbm, y0_hbm, y1_hbm,
                  pos_v, bufs, sem_g, sem_w):
    wid = lax.axis_index("s") * 2 + lax.axis_index("c")
    base = wid * TPW
    pltpu.sync_copy(pos_hbm.at[pl.ds(wid * 2 * TPW, 2 * TPW)], pos_v)
    for phase, y_hbm in ((0, y0_hbm), (1, y1_hbm)):
        gth = [None] * NG
        wrt = [None] * NG

        def _gather(g, phase=phase):
            idx = pos_v[pl.ds(phase * TPW + g * 16, 16)]
            return pltpu.async_copy(ys_hbm.at[idx], bufs.at[g % 2], sem_g)

        gth[0] = _gather(0)
        for g in range(NG):
            gth[g].wait()
            if g >= 1:
                wrt[g - 1].wait()
            if g + 1 < NG:
                gth[g + 1] = _gather(g + 1)
            wrt[g] = pltpu.async_copy(
                bufs.at[g % 2], y_hbm.at[pl.ds(base + g * 16, 16)], sem_w)
        wrt[NG - 1].wait()


def _combine(ys, pos):
    mesh = plsc.VectorSubcoreMesh(core_axis_name="c", subcore_axis_name="s")
    return pl.kernel(
        _combine_body,
        mesh=mesh,
        out_type=(
            jax.ShapeDtypeStruct((T, H), jnp.float32),
            jax.ShapeDtypeStruct((T, H), jnp.float32),
        ),
        scratch_types=[
            pltpu.VMEM((2 * TPW,), jnp.int32),
            pltpu.VMEM((2, 16, H), jnp.float32),
            pltpu.SemaphoreType.DMA,
            pltpu.SemaphoreType.DMA,
        ],
    )(ys, pos)


def kernel(hidden_states, gate_w, gate_ws, up_ws, down_ws,
           shared_gate_w, shared_up_w, shared_down_w):
    metat, w01, offr, be3, xbf = _router(hidden_states, gate_w)
    offr_flat = offr.reshape(E * 16)
    be = be3.reshape(NB)
    xs, pos = _dispatch(hidden_states, metat, offr_flat)
    ys = _mlp(be, xs, gate_ws, up_ws, down_ws)
    y0, y1 = _combine(ys, pos)
    return _shared(xbf, shared_gate_w, shared_up_w, shared_down_w,
                   y0, y1, w01)


# speedup vs baseline: 1.2055x; 1.0412x over previous
import functools

import jax
import jax.numpy as jnp
from jax import lax
from jax.experimental import pallas as pl
from jax.experimental.pallas import tpu as pltpu
from jax.experimental.pallas import tpu_sc as plsc

H = 2048
E = 8
F = 512
TOPK = 2
NGROUP = 4
GSZ = E // NGROUP
TOPKG = 2
SF = 1024
T = 2048

BLK = 256
NB = 24
NS = NB * BLK

TC = 8
CT = T // TC

NTILE = 32
TPW = T // NTILE
NG = TPW // 16


def _router_body(x_ref, gw_ref, metat_ref, w01_ref, offr_ref, be_ref,
                 xbf_ref, base_ref):
    c = pl.program_id(0)

    @pl.when(c == 0)
    def _():
        base_ref[...] = jnp.zeros_like(base_ref)

    x = x_ref[...]
    xbf_ref[...] = x.astype(jnp.bfloat16)
    gw = gw_ref[...]
    logits = lax.dot_general(
        x, gw, (((1,), (1,)), ((), ())), preferred_element_type=jnp.float32)
    m = jnp.max(logits, axis=-1, keepdims=True)
    ex = jnp.exp(logits - m)
    scores = ex / jnp.sum(ex, axis=-1, keepdims=True)

    s_sw = jnp.concatenate(
        [scores[:, i ^ 1:(i ^ 1) + 1] for i in range(E)], axis=1)
    gexp = jnp.maximum(scores, s_sw)

    ga = gexp[:, :, None]
    gb = gexp[:, None, :]
    ie = lax.broadcasted_iota(jnp.int32, (CT, E, E), 1)
    ij = lax.broadcasted_iota(jnp.int32, (CT, E, E), 2)
    gje = lax.shift_right_logical(ie, 1)
    gjj = lax.shift_right_logical(ij, 1)
    gbeat = (gb > ga) | ((gb == ga) & (gjj < gje))
    grank = jnp.sum(jnp.where(gbeat, 0.5, 0.0), axis=2)
    m = jnp.where(grank < float(TOPKG), scores, 0.0)

    ebeat = ((m[:, None, :] > m[:, :, None])
             | ((m[:, None, :] == m[:, :, None]) & (ij < ie)))
    rank8 = jnp.sum(jnp.where(ebeat, 1.0, 0.0), axis=2)

    sel0 = (rank8 == 0.0).astype(jnp.float32)
    sel1 = (rank8 == 1.0).astype(jnp.float32)
    iota8 = lax.broadcasted_iota(jnp.int32, (CT, E), 1).astype(jnp.float32)
    e0 = jnp.sum(sel0 * iota8, axis=1, keepdims=True)
    e1 = jnp.sum(sel1 * iota8, axis=1, keepdims=True)
    w0 = jnp.sum(sel0 * m, axis=1, keepdims=True)
    w1 = jnp.sum(sel1 * m, axis=1, keepdims=True)

    oh = sel0 + sel1
    ii = lax.broadcasted_iota(jnp.int32, (CT, CT), 0)
    jj = lax.broadcasted_iota(jnp.int32, (CT, CT), 1)
    lexc = jnp.where(ii > jj, 1.0, 0.0)
    within = lax.dot_general(
        lexc, oh, (((1,), (0,)), ((), ())),
        preferred_element_type=jnp.float32)
    rank_te = within + base_ref[...]
    r0 = jnp.sum(sel0 * rank_te, axis=1, keepdims=True)
    r1 = jnp.sum(sel1 * rank_te, axis=1, keepdims=True)

    zero = jnp.zeros_like(e0)
    m8 = jnp.concatenate([e0, e1, r0, r1, w0, w1, zero, zero], axis=1)
    ident = jnp.where(ii == jj, 1.0, 0.0)
    metat_ref[...] = lax.dot_general(
        m8, ident, (((0,), (0,)), ((), ())),
        preferred_element_type=jnp.float32,
        precision=lax.Precision.HIGHEST)

    w01_ref[...] = jnp.concatenate(
        [jnp.broadcast_to(w0, (CT, 16)), jnp.broadcast_to(w1, (CT, 16))],
        axis=1)

    base_ref[...] += jnp.sum(oh, axis=0, keepdims=True)

    @pl.when(c == TC - 1)
    def _():
        cnt = base_ref[...]
        padblk = jnp.floor((cnt + float(BLK - 1)) * (1.0 / BLK))
        offs = []
        run = jnp.zeros_like(padblk[:, 0:1])
        ends = []
        for e in range(E):
            offs.append(run * float(BLK))
            run = run + padblk[:, e:e + 1]
            ends.append(run)
        off2 = jnp.concatenate(offs, axis=1)
        i8a = lax.broadcasted_iota(jnp.int32, (E, E), 0)
        i8b = lax.broadcasted_iota(jnp.int32, (E, E), 1)
        ident8 = jnp.where(i8a == i8b, 1.0, 0.0)
        offcol = lax.dot_general(
            ident8, off2, (((1,), (1,)), ((), ())),
            preferred_element_type=jnp.float32,
            precision=lax.Precision.HIGHEST)
        offr_ref[...] = lax.dot_general(
            offcol, jnp.ones((1, 16), jnp.float32),
            (((1,), (0,)), ((), ())),
            preferred_element_type=jnp.float32,
            precision=lax.Precision.HIGHEST)
        bi = lax.broadcasted_iota(jnp.int32, (1, NB), 1).astype(jnp.float32)
        be = sum(jnp.where(bi >= ends[e], 1.0, 0.0) for e in range(E))
        be = jnp.minimum(be, float(E - 1))
        be_ref[...] = be.astype(jnp.int32).reshape(1, 1, NB)


def _router(x, gate_w):
    return pl.pallas_call(
        _router_body,
        grid=(TC,),
        in_specs=[
            pl.BlockSpec((CT, H), lambda c: (c, 0)),
            pl.BlockSpec((E, H), lambda c: (0, 0)),
        ],
        out_specs=(
            pl.BlockSpec((E, CT), lambda c: (0, c)),
            pl.BlockSpec((CT, 32), lambda c: (c, 0)),
            pl.BlockSpec((E, 16), lambda c: (0, 0)),
            pl.BlockSpec((1, 1, NB), lambda c: (0, 0, 0)),
            pl.BlockSpec((CT, H), lambda c: (c, 0)),
        ),
        out_shape=(
            jax.ShapeDtypeStruct((E, T), jnp.float32),
            jax.ShapeDtypeStruct((T, 32), jnp.float32),
            jax.ShapeDtypeStruct((E, 16), jnp.float32),
            jax.ShapeDtypeStruct((1, 1, NB), jnp.int32),
            jax.ShapeDtypeStruct((T, H), jnp.bfloat16),
        ),
        scratch_shapes=[pltpu.VMEM((1, E), jnp.float32)],
    )(x, gate_w)


def _dispatch_body(x_hbm, metat_hbm, offr_hbm, xs_hbm, pos_hbm,
                   mt_v, off_v, pos_v, rows_v, sem_l, sem_s):
    wid = lax.axis_index("s") * 2 + lax.axis_index("c")
    base = wid * TPW
    for r in range(4):
        pltpu.sync_copy(metat_hbm.at[r, pl.ds(base, TPW)],
                        mt_v.at[pl.ds(r * TPW, TPW)])
    pltpu.sync_copy(offr_hbm, off_v)
    posvecs = []
    for g in range(NG):
        ev0 = mt_v[pl.ds(0 * TPW + g * 16, 16)]
        ev1 = mt_v[pl.ds(1 * TPW + g * 16, 16)]
        p0 = mt_v[pl.ds(2 * TPW + g * 16, 16)]
        p1 = mt_v[pl.ds(3 * TPW + g * 16, 16)]
        for e in range(E):
            ov = off_v[pl.ds(e * 16, 16)]
            fe = float(e)
            p0 = p0 + jnp.where(ev0 == fe, ov, 0.0)
            p1 = p1 + jnp.where(ev1 == fe, ov, 0.0)
        i0 = p0.astype(jnp.int32)
        i1 = p1.astype(jnp.int32)
        pos_v[pl.ds(g * 16, 16)] = i0
        pos_v[pl.ds(TPW + g * 16, 16)] = i1
        posvecs.append((i0, i1))
    pltpu.sync_copy(pos_v, pos_hbm.at[pl.ds(wid * 2 * TPW, 2 * TPW)])
    loads = [None] * NG
    scts = [None] * NG

    def _load(g):
        return pltpu.async_copy(
            x_hbm.at[pl.ds(base + g * 16, 16)], rows_v.at[g % 2], sem_l)

    loads[0] = _load(0)
    for g in range(NG):
        loads[g].wait()
        if g >= 1:
            scts[g - 1][0].wait()
            scts[g - 1][1].wait()
        if g + 1 < NG:
            loads[g + 1] = _load(g + 1)
        i0, i1 = posvecs[g]
        scts[g] = (
            pltpu.async_copy(rows_v.at[g % 2], xs_hbm.at[i0], sem_s),
            pltpu.async_copy(rows_v.at[g % 2], xs_hbm.at[i1], sem_s),
        )
    scts[NG - 1][0].wait()
    scts[NG - 1][1].wait()


def _dispatch(x, metat, offr_flat):
    mesh = plsc.VectorSubcoreMesh(core_axis_name="c", subcore_axis_name="s")
    return pl.kernel(
        _dispatch_body,
        mesh=mesh,
        out_type=(
            jax.ShapeDtypeStruct((NS, H), jnp.float32),
            jax.ShapeDtypeStruct((2 * T,), jnp.int32),
        ),
        scratch_types=[
            pltpu.VMEM((4 * TPW,), jnp.float32),
            pltpu.VMEM((E * 16,), jnp.float32),
            pltpu.VMEM((2 * TPW,), jnp.int32),
            pltpu.VMEM((2, 16, H), jnp.float32),
            pltpu.SemaphoreType.DMA,
            pltpu.SemaphoreType.DMA,
        ],
    )(x, metat, offr_flat)


def _mlp_body(be_ref, xs_ref, wg_ref, wu_ref, wd_ref, ys_ref):
    xb = xs_ref[...].astype(jnp.bfloat16)
    wg = wg_ref[0].astype(jnp.bfloat16)
    wu = wu_ref[0].astype(jnp.bfloat16)
    wd = wd_ref[0].astype(jnp.bfloat16)
    g = lax.dot_general(xb, wg, (((1,), (1,)), ((), ())),
                        preferred_element_type=jnp.float32)
    u = lax.dot_general(xb, wu, (((1,), (1,)), ((), ())),
                        preferred_element_type=jnp.float32)
    h = (g * jax.nn.sigmoid(g) * u).astype(jnp.bfloat16)
    ys_ref[...] = lax.dot_general(h, wd, (((1,), (1,)), ((), ())),
                                  preferred_element_type=jnp.float32)


def _mlp(be, xs, gate_ws, up_ws, down_ws):
    return pl.pallas_call(
        _mlp_body,
        grid_spec=pltpu.PrefetchScalarGridSpec(
            num_scalar_prefetch=1,
            grid=(NB,),
            in_specs=[
                pl.BlockSpec((BLK, H), lambda b, be: (b, 0)),
                pl.BlockSpec((1, F, H), lambda b, be: (be[b], 0, 0)),
                pl.BlockSpec((1, F, H), lambda b, be: (be[b], 0, 0)),
                pl.BlockSpec((1, H, F), lambda b, be: (be[b], 0, 0)),
            ],
            out_specs=pl.BlockSpec((BLK, H), lambda b, be: (b, 0)),
        ),
        out_shape=jax.ShapeDtypeStruct((NS, H), jnp.float32),
    )(be, xs, gate_ws, up_ws, down_ws)


SBT = 8
SBR = T // SBT


def _shared_body(xbf_ref, wg_ref, wu_ref, wd_ref, y0_ref, y1_ref, w01_ref,
                 out_ref):
    xb = xbf_ref[...]
    wg = wg_ref[...].astype(jnp.bfloat16)
    wu = wu_ref[...].astype(jnp.bfloat16)
    wd = wd_ref[...].astype(jnp.bfloat16)
    g = lax.dot_general(xb, wg, (((1,), (1,)), ((), ())),
                        preferred_element_type=jnp.float32)
    u = lax.dot_general(xb, wu, (((1,), (1,)), ((), ())),
                        preferred_element_type=jnp.float32)
    h = (g * jax.nn.sigmoid(g) * u).astype(jnp.bfloat16)
    shared = lax.dot_general(h, wd, (((1,), (1,)), ((), ())),
                             preferred_element_type=jnp.float32)
    w0 = w01_ref[:, 0:1]
    w1 = w01_ref[:, 16:17]
    out_ref[...] = (shared
                    + w0 * y0_ref[...].astype(jnp.float32)
                    + w1 * y1_ref[...].astype(jnp.float32))


def _shared(xbf, sg_w, su_w, sd_w, y0, y1, w01):
    return pl.pallas_call(
        _shared_body,
        grid=(SBT,),
        in_specs=[
            pl.BlockSpec((SBR, H), lambda i: (i, 0)),
            pl.BlockSpec((SF, H), lambda i: (0, 0)),
            pl.BlockSpec((SF, H), lambda i: (0, 0)),
            pl.BlockSpec((H, SF), lambda i: (0, 0)),
            pl.BlockSpec((SBR, H), lambda i: (i, 0)),
            pl.BlockSpec((SBR, H), lambda i: (i, 0)),
            pl.BlockSpec((SBR, 32), lambda i: (i, 0)),
        ],
        out_specs=pl.BlockSpec((SBR, H), lambda i: (i, 0)),
        out_shape=jax.ShapeDtypeStruct((T, H), jnp.float32),
    )(xbf, sg_w, su_w, sd_w, y0, y1, w01)


def _combine_body(ys_hbm, pos_hbm, y0_hbm, y1_hbm,
                  pos_v, bufs, sem_g, sem_w):
    wid = lax.axis_index("s") * 2 + lax.axis_index("c")
    base = wid * TPW
    pltpu.sync_copy(pos_hbm.at[pl.ds(wid * 2 * TPW, 2 * TPW)], pos_v)
    for phase, y_hbm in ((0, y0_hbm), (1, y1_hbm)):
        gth = [None] * NG
        wrt = [None] * NG

        def _gather(g, phase=phase):
            idx = pos_v[pl.ds(phase * TPW + g * 16, 16)]
            return pltpu.async_copy(ys_hbm.at[idx], bufs.at[g % 2], sem_g)

        gth[0] = _gather(0)
        for g in range(NG):
            gth[g].wait()
            if g >= 1:
                wrt[g - 1].wait()
            if g + 1 < NG:
                gth[g + 1] = _gather(g + 1)
            wrt[g] = pltpu.async_copy(
                bufs.at[g % 2], y_hbm.at[pl.ds(base + g * 16, 16)], sem_w)
        wrt[NG - 1].wait()


def _combine(ys, pos):
    mesh = plsc.VectorSubcoreMesh(core_axis_name="c", subcore_axis_name="s")
    return pl.kernel(
        _combine_body,
        mesh=mesh,
        out_type=(
            jax.ShapeDtypeStruct((T, H), jnp.float32),
            jax.ShapeDtypeStruct((T, H), jnp.float32),
        ),
        scratch_types=[
            pltpu.VMEM((2 * TPW,), jnp.int32),
            pltpu.VMEM((2, 16, H), jnp.float32),
            pltpu.SemaphoreType.DMA,
            pltpu.SemaphoreType.DMA,
        ],
    )(ys, pos)


def kernel(hidden_states, gate_w, gate_ws, up_ws, down_ws,
           shared_gate_w, shared_up_w, shared_down_w):
    metat, w01, offr, be3, xbf = _router(hidden_states, gate_w)
    offr_flat = offr.reshape(E * 16)
    be = be3.reshape(NB)
    xs, pos = _dispatch(hidden_states, metat, offr_flat)
    ys = _mlp(be, xs, gate_ws, up_ws, down_ws)
    y0, y1 = _combine(ys, pos)
    return _shared(xbf, shared_gate_w, shared_up_w, shared_down_w,
                   y0, y1, w01)


# speedup vs baseline: 1.4146x; 1.1735x over previous
import functools

import jax
import jax.numpy as jnp
from jax import lax
from jax.experimental import pallas as pl
from jax.experimental.pallas import tpu as pltpu
from jax.experimental.pallas import tpu_sc as plsc

H = 2048
E = 8
F = 512
TOPK = 2
NGROUP = 4
GSZ = E // NGROUP
TOPKG = 2
SF = 1024
T = 2048

BLK = 256
NB = 24
NS = NB * BLK

TC = 8
CT = T // TC

NTILE = 32
TPW = T // NTILE
NG = TPW // 16


def _router_body(x_ref, gw_ref, metat_ref, w01_ref, offr_ref, be_ref,
                 xbf_ref, base_ref):
    c = pl.program_id(0)

    @pl.when(c == 0)
    def _():
        base_ref[...] = jnp.zeros_like(base_ref)

    x = x_ref[...]
    xbf_ref[...] = x.astype(jnp.bfloat16)
    gw = gw_ref[...]
    logits = lax.dot_general(
        x, gw, (((1,), (1,)), ((), ())), preferred_element_type=jnp.float32)
    m = jnp.max(logits, axis=-1, keepdims=True)
    ex = jnp.exp(logits - m)
    scores = ex / jnp.sum(ex, axis=-1, keepdims=True)

    i8 = lax.broadcasted_iota(jnp.int32, (E, E * E), 0)
    p8 = lax.broadcasted_iota(jnp.int32, (E, E * E), 1)
    gmat = jnp.where(i8 == lax.shift_right_logical(p8, 3), 1.0, 0.0)
    rmat = jnp.where(i8 == (p8 & 7), 1.0, 0.0)
    p64 = lax.broadcasted_iota(jnp.int32, (E * E, E), 0)
    e8 = lax.broadcasted_iota(jnp.int32, (E * E, E), 1)
    hmat = jnp.where((p64 & 7) == e8, 1.0, 0.0)
    pl_ = lax.broadcasted_iota(jnp.int32, (CT, E * E), 1)
    pj = lax.shift_right_logical(pl_, 3)
    pe = pl_ & 7

    def _dotn(a, b):
        return lax.dot_general(a, b, (((1,), (0,)), ((), ())),
                               preferred_element_type=jnp.float32)

    def _doth(a, b):
        return lax.dot_general(a, b, (((1,), (0,)), ((), ())),
                               preferred_element_type=jnp.float32,
                               precision=lax.Precision.HIGHEST)

    s_sw = jnp.concatenate(
        [scores[:, i ^ 1:(i ^ 1) + 1] for i in range(E)], axis=1)
    gexp = jnp.maximum(scores, s_sw)

    gb_ = _doth(gexp, gmat)
    gc_ = _doth(gexp, rmat)
    gbeat = (gb_ > gc_) | ((gb_ == gc_)
                           & (lax.shift_right_logical(pj, 1)
                              < lax.shift_right_logical(pe, 1)))
    grank = _dotn(jnp.where(gbeat, 1.0, 0.0), hmat) * 0.5
    msk = jnp.where(grank < float(TOPKG), scores, 0.0)

    eb_ = _doth(msk, gmat)
    ec_ = _doth(msk, rmat)
    ebeat = (eb_ > ec_) | ((eb_ == ec_) & (pj < pe))
    rank8 = _dotn(jnp.where(ebeat, 1.0, 0.0), hmat)

    sel0 = (rank8 == 0.0).astype(jnp.float32)
    sel1 = (rank8 == 1.0).astype(jnp.float32)

    oh = sel0 + sel1
    ii = lax.broadcasted_iota(jnp.int32, (CT, CT), 0)
    jj = lax.broadcasted_iota(jnp.int32, (CT, CT), 1)
    lexc = jnp.where(ii > jj, 1.0, 0.0)
    within = lax.dot_general(
        lexc, oh, (((1,), (0,)), ((), ())),
        preferred_element_type=jnp.float32)
    rank_te = within + base_ref[...]

    iota8c = lax.broadcasted_iota(
        jnp.int32, (E, 1), 0).astype(jnp.float32)
    ones8 = jnp.ones((E, 1), jnp.float32)

    e0 = _doth(sel0, iota8c)
    e1 = _doth(sel1, iota8c)
    w0 = _doth(sel0 * msk, ones8)
    w1 = _doth(sel1 * msk, ones8)
    r0 = _doth(sel0 * rank_te, ones8)
    r1 = _doth(sel1 * rank_te, ones8)

    zero = jnp.zeros_like(e0)
    m8 = jnp.concatenate([e0, e1, r0, r1, w0, w1, zero, zero], axis=1)
    ident = jnp.where(ii == jj, 1.0, 0.0)
    metat_ref[...] = lax.dot_general(
        m8, ident, (((0,), (0,)), ((), ())),
        preferred_element_type=jnp.float32,
        precision=lax.Precision.HIGHEST)

    w01_ref[...] = jnp.concatenate(
        [jnp.broadcast_to(w0, (CT, 16)), jnp.broadcast_to(w1, (CT, 16))],
        axis=1)

    base_ref[...] += jnp.sum(oh, axis=0, keepdims=True)

    @pl.when(c == TC - 1)
    def _():
        cnt = base_ref[...]
        padblk = jnp.floor((cnt + float(BLK - 1)) * (1.0 / BLK))
        offs = []
        run = jnp.zeros_like(padblk[:, 0:1])
        ends = []
        for e in range(E):
            offs.append(run * float(BLK))
            run = run + padblk[:, e:e + 1]
            ends.append(run)
        off2 = jnp.concatenate(offs, axis=1)
        i8a = lax.broadcasted_iota(jnp.int32, (E, E), 0)
        i8b = lax.broadcasted_iota(jnp.int32, (E, E), 1)
        ident8 = jnp.where(i8a == i8b, 1.0, 0.0)
        offcol = lax.dot_general(
            ident8, off2, (((1,), (1,)), ((), ())),
            preferred_element_type=jnp.float32,
            precision=lax.Precision.HIGHEST)
        offr_ref[...] = lax.dot_general(
            offcol, jnp.ones((1, 16), jnp.float32),
            (((1,), (0,)), ((), ())),
            preferred_element_type=jnp.float32,
            precision=lax.Precision.HIGHEST)
        bi = lax.broadcasted_iota(jnp.int32, (1, NB), 1).astype(jnp.float32)
        be = sum(jnp.where(bi >= ends[e], 1.0, 0.0) for e in range(E))
        be = jnp.minimum(be, float(E - 1))
        be_ref[...] = be.astype(jnp.int32).reshape(1, 1, NB)


def _router(x, gate_w):
    return pl.pallas_call(
        _router_body,
        grid=(TC,),
        in_specs=[
            pl.BlockSpec((CT, H), lambda c: (c, 0)),
            pl.BlockSpec((E, H), lambda c: (0, 0)),
        ],
        out_specs=(
            pl.BlockSpec((E, CT), lambda c: (0, c)),
            pl.BlockSpec((CT, 32), lambda c: (c, 0)),
            pl.BlockSpec((E, 16), lambda c: (0, 0)),
            pl.BlockSpec((1, 1, NB), lambda c: (0, 0, 0)),
            pl.BlockSpec((CT, H), lambda c: (c, 0)),
        ),
        out_shape=(
            jax.ShapeDtypeStruct((E, T), jnp.float32),
            jax.ShapeDtypeStruct((T, 32), jnp.float32),
            jax.ShapeDtypeStruct((E, 16), jnp.float32),
            jax.ShapeDtypeStruct((1, 1, NB), jnp.int32),
            jax.ShapeDtypeStruct((T, H), jnp.bfloat16),
        ),
        scratch_shapes=[pltpu.VMEM((1, E), jnp.float32)],
    )(x, gate_w)


def _dispatch_body(x_hbm, metat_hbm, offr_hbm, xs_hbm, pos_hbm,
                   mt_v, off_v, pos_v, rows_v, sem_l, sem_s):
    wid = lax.axis_index("s") * 2 + lax.axis_index("c")
    base = wid * TPW
    for r in range(4):
        pltpu.sync_copy(metat_hbm.at[r, pl.ds(base, TPW)],
                        mt_v.at[pl.ds(r * TPW, TPW)])
    pltpu.sync_copy(offr_hbm, off_v)
    posvecs = []
    for g in range(NG):
        ev0 = mt_v[pl.ds(0 * TPW + g * 16, 16)]
        ev1 = mt_v[pl.ds(1 * TPW + g * 16, 16)]
        p0 = mt_v[pl.ds(2 * TPW + g * 16, 16)]
        p1 = mt_v[pl.ds(3 * TPW + g * 16, 16)]
        for e in range(E):
            ov = off_v[pl.ds(e * 16, 16)]
            fe = float(e)
            p0 = p0 + jnp.where(ev0 == fe, ov, 0.0)
            p1 = p1 + jnp.where(ev1 == fe, ov, 0.0)
        i0 = p0.astype(jnp.int32)
        i1 = p1.astype(jnp.int32)
        pos_v[pl.ds(g * 16, 16)] = i0
        pos_v[pl.ds(TPW + g * 16, 16)] = i1
        posvecs.append((i0, i1))
    pltpu.sync_copy(pos_v, pos_hbm.at[pl.ds(wid * 2 * TPW, 2 * TPW)])
    loads = [None] * NG
    scts = [None] * NG

    def _load(g):
        return pltpu.async_copy(
            x_hbm.at[pl.ds(base + g * 16, 16)], rows_v.at[g % 2], sem_l)

    loads[0] = _load(0)
    for g in range(NG):
        loads[g].wait()
        if g >= 1:
            scts[g - 1][0].wait()
            scts[g - 1][1].wait()
        if g + 1 < NG:
            loads[g + 1] = _load(g + 1)
        i0, i1 = posvecs[g]
        scts[g] = (
            pltpu.async_copy(rows_v.at[g % 2], xs_hbm.at[i0], sem_s),
            pltpu.async_copy(rows_v.at[g % 2], xs_hbm.at[i1], sem_s),
        )
    scts[NG - 1][0].wait()
    scts[NG - 1][1].wait()


def _dispatch(x, metat, offr_flat):
    mesh = plsc.VectorSubcoreMesh(core_axis_name="c", subcore_axis_name="s")
    return pl.kernel(
        _dispatch_body,
        mesh=mesh,
        out_type=(
            jax.ShapeDtypeStruct((NS, H), jnp.float32),
            jax.ShapeDtypeStruct((2 * T,), jnp.int32),
        ),
        scratch_types=[
            pltpu.VMEM((4 * TPW,), jnp.float32),
            pltpu.VMEM((E * 16,), jnp.float32),
            pltpu.VMEM((2 * TPW,), jnp.int32),
            pltpu.VMEM((2, 16, H), jnp.float32),
            pltpu.SemaphoreType.DMA,
            pltpu.SemaphoreType.DMA,
        ],
    )(x, metat, offr_flat)


def _mlp_body(be_ref, xs_ref, wg_ref, wu_ref, wd_ref, ys_ref):
    xb = xs_ref[...].astype(jnp.bfloat16)
    wg = wg_ref[0].astype(jnp.bfloat16)
    wu = wu_ref[0].astype(jnp.bfloat16)
    wd = wd_ref[0].astype(jnp.bfloat16)
    g = lax.dot_general(xb, wg, (((1,), (1,)), ((), ())),
                        preferred_element_type=jnp.float32)
    u = lax.dot_general(xb, wu, (((1,), (1,)), ((), ())),
                        preferred_element_type=jnp.float32)
    h = (g * jax.nn.sigmoid(g) * u).astype(jnp.bfloat16)
    ys_ref[...] = lax.dot_general(h, wd, (((1,), (1,)), ((), ())),
                                  preferred_element_type=jnp.float32)


def _mlp(be, xs, gate_ws, up_ws, down_ws):
    return pl.pallas_call(
        _mlp_body,
        grid_spec=pltpu.PrefetchScalarGridSpec(
            num_scalar_prefetch=1,
            grid=(NB,),
            in_specs=[
                pl.BlockSpec((BLK, H), lambda b, be: (b, 0)),
                pl.BlockSpec((1, F, H), lambda b, be: (be[b], 0, 0)),
                pl.BlockSpec((1, F, H), lambda b, be: (be[b], 0, 0)),
                pl.BlockSpec((1, H, F), lambda b, be: (be[b], 0, 0)),
            ],
            out_specs=pl.BlockSpec((BLK, H), lambda b, be: (b, 0)),
        ),
        out_shape=jax.ShapeDtypeStruct((NS, H), jnp.float32),
    )(be, xs, gate_ws, up_ws, down_ws)


SBT = 8
SBR = T // SBT


def _shared_body(xbf_ref, wg_ref, wu_ref, wd_ref, y0_ref, y1_ref, w01_ref,
                 out_ref):
    xb = xbf_ref[...]
    wg = wg_ref[...].astype(jnp.bfloat16)
    wu = wu_ref[...].astype(jnp.bfloat16)
    wd = wd_ref[...].astype(jnp.bfloat16)
    g = lax.dot_general(xb, wg, (((1,), (1,)), ((), ())),
                        preferred_element_type=jnp.float32)
    u = lax.dot_general(xb, wu, (((1,), (1,)), ((), ())),
                        preferred_element_type=jnp.float32)
    h = (g * jax.nn.sigmoid(g) * u).astype(jnp.bfloat16)
    shared = lax.dot_general(h, wd, (((1,), (1,)), ((), ())),
                             preferred_element_type=jnp.float32)
    w0 = w01_ref[:, 0:1]
    w1 = w01_ref[:, 16:17]
    out_ref[...] = (shared
                    + w0 * y0_ref[...].astype(jnp.float32)
                    + w1 * y1_ref[...].astype(jnp.float32))


def _shared(xbf, sg_w, su_w, sd_w, y0, y1, w01):
    return pl.pallas_call(
        _shared_body,
        grid=(SBT,),
        in_specs=[
            pl.BlockSpec((SBR, H), lambda i: (i, 0)),
            pl.BlockSpec((SF, H), lambda i: (0, 0)),
            pl.BlockSpec((SF, H), lambda i: (0, 0)),
            pl.BlockSpec((H, SF), lambda i: (0, 0)),
            pl.BlockSpec((SBR, H), lambda i: (i, 0)),
            pl.BlockSpec((SBR, H), lambda i: (i, 0)),
            pl.BlockSpec((SBR, 32), lambda i: (i, 0)),
        ],
        out_specs=pl.BlockSpec((SBR, H), lambda i: (i, 0)),
        out_shape=jax.ShapeDtypeStruct((T, H), jnp.float32),
    )(xbf, sg_w, su_w, sd_w, y0, y1, w01)


def _combine_body(ys_hbm, pos_hbm, y0_hbm, y1_hbm,
                  pos_v, bufs, sem_g, sem_w):
    wid = lax.axis_index("s") * 2 + lax.axis_index("c")
    base = wid * TPW
    pltpu.sync_copy(pos_hbm.at[pl.ds(wid * 2 * TPW, 2 * TPW)], pos_v)
    for phase, y_hbm in ((0, y0_hbm), (1, y1_hbm)):
        gth = [None] * NG
        wrt = [None] * NG

        def _gather(g, phase=phase):
            idx = pos_v[pl.ds(phase * TPW + g * 16, 16)]
            return pltpu.async_copy(ys_hbm.at[idx], bufs.at[g % 2], sem_g)

        gth[0] = _gather(0)
        for g in range(NG):
            gth[g].wait()
            if g >= 1:
                wrt[g - 1].wait()
            if g + 1 < NG:
                gth[g + 1] = _gather(g + 1)
            wrt[g] = pltpu.async_copy(
                bufs.at[g % 2], y_hbm.at[pl.ds(base + g * 16, 16)], sem_w)
        wrt[NG - 1].wait()


def _combine(ys, pos):
    mesh = plsc.VectorSubcoreMesh(core_axis_name="c", subcore_axis_name="s")
    return pl.kernel(
        _combine_body,
        mesh=mesh,
        out_type=(
            jax.ShapeDtypeStruct((T, H), jnp.float32),
            jax.ShapeDtypeStruct((T, H), jnp.float32),
        ),
        scratch_types=[
            pltpu.VMEM((2 * TPW,), jnp.int32),
            pltpu.VMEM((2, 16, H), jnp.float32),
            pltpu.SemaphoreType.DMA,
            pltpu.SemaphoreType.DMA,
        ],
    )(ys, pos)


def kernel(hidden_states, gate_w, gate_ws, up_ws, down_ws,
           shared_gate_w, shared_up_w, shared_down_w):
    metat, w01, offr, be3, xbf = _router(hidden_states, gate_w)
    offr_flat = offr.reshape(E * 16)
    be = be3.reshape(NB)
    xs, pos = _dispatch(hidden_states, metat, offr_flat)
    ys = _mlp(be, xs, gate_ws, up_ws, down_ws)
    y0, y1 = _combine(ys, pos)
    return _shared(xbf, shared_gate_w, shared_up_w, shared_down_w,
                   y0, y1, w01)


# speedup vs baseline: 1.4180x; 1.0024x over previous
import functools

import jax
import jax.numpy as jnp
from jax import lax
from jax.experimental import pallas as pl
from jax.experimental.pallas import tpu as pltpu
from jax.experimental.pallas import tpu_sc as plsc

H = 2048
E = 8
F = 512
TOPK = 2
NGROUP = 4
GSZ = E // NGROUP
TOPKG = 2
SF = 1024
T = 2048

BLK = 256
NB = 24
NS = NB * BLK

TC = 8
CT = T // TC

NTILE = 32
TPW = T // NTILE
NG = TPW // 16


def _router_body(x_ref, gw_ref, metat_ref, w01_ref, offr_ref, be_ref,
                 base_ref):
    c = pl.program_id(0)

    @pl.when(c == 0)
    def _():
        base_ref[...] = jnp.zeros_like(base_ref)

    x = x_ref[...]
    gw = gw_ref[...]
    logits = lax.dot_general(
        x, gw, (((1,), (1,)), ((), ())), preferred_element_type=jnp.float32)
    m = jnp.max(logits, axis=-1, keepdims=True)
    ex = jnp.exp(logits - m)
    scores = ex / jnp.sum(ex, axis=-1, keepdims=True)

    i8 = lax.broadcasted_iota(jnp.int32, (E, E * E), 0)
    p8 = lax.broadcasted_iota(jnp.int32, (E, E * E), 1)
    gmat = jnp.where(i8 == lax.shift_right_logical(p8, 3), 1.0, 0.0)
    rmat = jnp.where(i8 == (p8 & 7), 1.0, 0.0)
    p64 = lax.broadcasted_iota(jnp.int32, (E * E, E), 0)
    e8 = lax.broadcasted_iota(jnp.int32, (E * E, E), 1)
    hmat = jnp.where((p64 & 7) == e8, 1.0, 0.0)
    pl_ = lax.broadcasted_iota(jnp.int32, (CT, E * E), 1)
    pj = lax.shift_right_logical(pl_, 3)
    pe = pl_ & 7

    def _dotn(a, b):
        return lax.dot_general(a, b, (((1,), (0,)), ((), ())),
                               preferred_element_type=jnp.float32)

    def _doth(a, b):
        return lax.dot_general(a, b, (((1,), (0,)), ((), ())),
                               preferred_element_type=jnp.float32,
                               precision=lax.Precision.HIGHEST)

    s_sw = jnp.concatenate(
        [scores[:, i ^ 1:(i ^ 1) + 1] for i in range(E)], axis=1)
    gexp = jnp.maximum(scores, s_sw)

    gb_ = _doth(gexp, gmat)
    gc_ = _doth(gexp, rmat)
    gbeat = (gb_ > gc_) | ((gb_ == gc_)
                           & (lax.shift_right_logical(pj, 1)
                              < lax.shift_right_logical(pe, 1)))
    grank = _dotn(jnp.where(gbeat, 1.0, 0.0), hmat) * 0.5
    msk = jnp.where(grank < float(TOPKG), scores, 0.0)

    eb_ = _doth(msk, gmat)
    ec_ = _doth(msk, rmat)
    ebeat = (eb_ > ec_) | ((eb_ == ec_) & (pj < pe))
    rank8 = _dotn(jnp.where(ebeat, 1.0, 0.0), hmat)

    sel0 = (rank8 == 0.0).astype(jnp.float32)
    sel1 = (rank8 == 1.0).astype(jnp.float32)

    oh = sel0 + sel1
    ii = lax.broadcasted_iota(jnp.int32, (CT, CT), 0)
    jj = lax.broadcasted_iota(jnp.int32, (CT, CT), 1)
    lexc = jnp.where(ii > jj, 1.0, 0.0)
    within = lax.dot_general(
        lexc, oh, (((1,), (0,)), ((), ())),
        preferred_element_type=jnp.float32)
    rank_te = within + base_ref[...]

    iota8c = lax.broadcasted_iota(
        jnp.int32, (E, 1), 0).astype(jnp.float32)
    ones8 = jnp.ones((E, 1), jnp.float32)

    e0 = _doth(sel0, iota8c)
    e1 = _doth(sel1, iota8c)
    w0 = _doth(sel0 * msk, ones8)
    w1 = _doth(sel1 * msk, ones8)
    r0 = _doth(sel0 * rank_te, ones8)
    r1 = _doth(sel1 * rank_te, ones8)

    zero = jnp.zeros_like(e0)
    m8 = jnp.concatenate([e0, e1, r0, r1, w0, w1, zero, zero], axis=1)
    ident = jnp.where(ii == jj, 1.0, 0.0)
    metat_ref[...] = lax.dot_general(
        m8, ident, (((0,), (0,)), ((), ())),
        preferred_element_type=jnp.float32,
        precision=lax.Precision.HIGHEST)

    w01_ref[...] = jnp.concatenate(
        [jnp.broadcast_to(w0, (CT, 16)), jnp.broadcast_to(w1, (CT, 16))],
        axis=1)

    base_ref[...] += jnp.sum(oh, axis=0, keepdims=True)

    @pl.when(c == TC - 1)
    def _():
        cnt = base_ref[...]
        padblk = jnp.floor((cnt + float(BLK - 1)) * (1.0 / BLK))
        offs = []
        run = jnp.zeros_like(padblk[:, 0:1])
        ends = []
        for e in range(E):
            offs.append(run * float(BLK))
            run = run + padblk[:, e:e + 1]
            ends.append(run)
        off2 = jnp.concatenate(offs, axis=1)
        i8a = lax.broadcasted_iota(jnp.int32, (E, E), 0)
        i8b = lax.broadcasted_iota(jnp.int32, (E, E), 1)
        ident8 = jnp.where(i8a == i8b, 1.0, 0.0)
        offcol = lax.dot_general(
            ident8, off2, (((1,), (1,)), ((), ())),
            preferred_element_type=jnp.float32,
            precision=lax.Precision.HIGHEST)
        offr_ref[...] = lax.dot_general(
            offcol, jnp.ones((1, 16), jnp.float32),
            (((1,), (0,)), ((), ())),
            preferred_element_type=jnp.float32,
            precision=lax.Precision.HIGHEST)
        bi = lax.broadcasted_iota(jnp.int32, (1, NB), 1).astype(jnp.float32)
        be = sum(jnp.where(bi >= ends[e], 1.0, 0.0) for e in range(E))
        be = jnp.minimum(be, float(E - 1))
        be_ref[...] = be.astype(jnp.int32).reshape(1, 1, NB)


def _router(x, gate_w):
    return pl.pallas_call(
        _router_body,
        grid=(TC,),
        in_specs=[
            pl.BlockSpec((CT, H), lambda c: (c, 0)),
            pl.BlockSpec((E, H), lambda c: (0, 0)),
        ],
        out_specs=(
            pl.BlockSpec((E, CT), lambda c: (0, c)),
            pl.BlockSpec((CT, 32), lambda c: (c, 0)),
            pl.BlockSpec((E, 16), lambda c: (0, 0)),
            pl.BlockSpec((1, 1, NB), lambda c: (0, 0, 0)),
        ),
        out_shape=(
            jax.ShapeDtypeStruct((E, T), jnp.float32),
            jax.ShapeDtypeStruct((T, 32), jnp.float32),
            jax.ShapeDtypeStruct((E, 16), jnp.float32),
            jax.ShapeDtypeStruct((1, 1, NB), jnp.int32),
        ),
        scratch_shapes=[pltpu.VMEM((1, E), jnp.float32)],
    )(x, gate_w)


def _dispatch_body(x_hbm, metat_hbm, offr_hbm, xs_hbm, pos_hbm,
                   mt_v, off_v, pos_v, rows_v, sem_l, sem_s):
    wid = lax.axis_index("s") * 2 + lax.axis_index("c")
    base = wid * TPW
    for r in range(4):
        pltpu.sync_copy(metat_hbm.at[r, pl.ds(base, TPW)],
                        mt_v.at[pl.ds(r * TPW, TPW)])
    pltpu.sync_copy(offr_hbm, off_v)
    posvecs = []
    for g in range(NG):
        ev0 = mt_v[pl.ds(0 * TPW + g * 16, 16)]
        ev1 = mt_v[pl.ds(1 * TPW + g * 16, 16)]
        p0 = mt_v[pl.ds(2 * TPW + g * 16, 16)]
        p1 = mt_v[pl.ds(3 * TPW + g * 16, 16)]
        for e in range(E):
            ov = off_v[pl.ds(e * 16, 16)]
            fe = float(e)
            p0 = p0 + jnp.where(ev0 == fe, ov, 0.0)
            p1 = p1 + jnp.where(ev1 == fe, ov, 0.0)
        i0 = p0.astype(jnp.int32)
        i1 = p1.astype(jnp.int32)
        pos_v[pl.ds(g * 16, 16)] = i0
        pos_v[pl.ds(TPW + g * 16, 16)] = i1
        posvecs.append((i0, i1))
    pltpu.sync_copy(pos_v, pos_hbm.at[pl.ds(wid * 2 * TPW, 2 * TPW)])
    loads = [None] * NG
    scts = [None] * NG

    def _load(g):
        return pltpu.async_copy(
            x_hbm.at[pl.ds(base + g * 16, 16)], rows_v.at[g % 2], sem_l)

    loads[0] = _load(0)
    for g in range(NG):
        loads[g].wait()
        if g >= 1:
            scts[g - 1][0].wait()
            scts[g - 1][1].wait()
        if g + 1 < NG:
            loads[g + 1] = _load(g + 1)
        i0, i1 = posvecs[g]
        scts[g] = (
            pltpu.async_copy(rows_v.at[g % 2], xs_hbm.at[i0], sem_s),
            pltpu.async_copy(rows_v.at[g % 2], xs_hbm.at[i1], sem_s),
        )
    scts[NG - 1][0].wait()
    scts[NG - 1][1].wait()


def _dispatch(x, metat, offr_flat):
    mesh = plsc.VectorSubcoreMesh(core_axis_name="c", subcore_axis_name="s")
    return pl.kernel(
        _dispatch_body,
        mesh=mesh,
        out_type=(
            jax.ShapeDtypeStruct((NS, H), jnp.float32),
            jax.ShapeDtypeStruct((2 * T,), jnp.int32),
        ),
        scratch_types=[
            pltpu.VMEM((4 * TPW,), jnp.float32),
            pltpu.VMEM((E * 16,), jnp.float32),
            pltpu.VMEM((2 * TPW,), jnp.int32),
            pltpu.VMEM((2, 16, H), jnp.float32),
            pltpu.SemaphoreType.DMA,
            pltpu.SemaphoreType.DMA,
        ],
    )(x, metat, offr_flat)


def _mlp_body(be_ref, xs_ref, wg_ref, wu_ref, wd_ref, ys_ref):
    xb = xs_ref[...]
    wg = wg_ref[0]
    wu = wu_ref[0]
    wd = wd_ref[0]
    g = lax.dot_general(xb, wg, (((1,), (1,)), ((), ())),
                        preferred_element_type=jnp.float32)
    u = lax.dot_general(xb, wu, (((1,), (1,)), ((), ())),
                        preferred_element_type=jnp.float32)
    h = g * jax.nn.sigmoid(g) * u
    ys_ref[...] = lax.dot_general(h, wd, (((1,), (1,)), ((), ())),
                                  preferred_element_type=jnp.float32)


def _mlp(be, xs, gate_ws, up_ws, down_ws):
    return pl.pallas_call(
        _mlp_body,
        grid_spec=pltpu.PrefetchScalarGridSpec(
            num_scalar_prefetch=1,
            grid=(NB,),
            in_specs=[
                pl.BlockSpec((BLK, H), lambda b, be: (b, 0)),
                pl.BlockSpec((1, F, H), lambda b, be: (be[b], 0, 0)),
                pl.BlockSpec((1, F, H), lambda b, be: (be[b], 0, 0)),
                pl.BlockSpec((1, H, F), lambda b, be: (be[b], 0, 0)),
            ],
            out_specs=pl.BlockSpec((BLK, H), lambda b, be: (b, 0)),
        ),
        out_shape=jax.ShapeDtypeStruct((NS, H), jnp.float32),
    )(be, xs, gate_ws, up_ws, down_ws)


SBT = 8
SBR = T // SBT


def _shared_body(x_ref, wg_ref, wu_ref, wd_ref, y0_ref, y1_ref, w01_ref,
                 out_ref):
    xb = x_ref[...]
    wg = wg_ref[...]
    wu = wu_ref[...]
    wd = wd_ref[...]
    g = lax.dot_general(xb, wg, (((1,), (1,)), ((), ())),
                        preferred_element_type=jnp.float32)
    u = lax.dot_general(xb, wu, (((1,), (1,)), ((), ())),
                        preferred_element_type=jnp.float32)
    h = g * jax.nn.sigmoid(g) * u
    shared = lax.dot_general(h, wd, (((1,), (1,)), ((), ())),
                             preferred_element_type=jnp.float32)
    w0 = w01_ref[:, 0:1]
    w1 = w01_ref[:, 16:17]
    out_ref[...] = (shared
                    + w0 * y0_ref[...].astype(jnp.float32)
                    + w1 * y1_ref[...].astype(jnp.float32))


def _shared(x, sg_w, su_w, sd_w, y0, y1, w01):
    return pl.pallas_call(
        _shared_body,
        grid=(SBT,),
        in_specs=[
            pl.BlockSpec((SBR, H), lambda i: (i, 0)),
            pl.BlockSpec((SF, H), lambda i: (0, 0)),
            pl.BlockSpec((SF, H), lambda i: (0, 0)),
            pl.BlockSpec((H, SF), lambda i: (0, 0)),
            pl.BlockSpec((SBR, H), lambda i: (i, 0)),
            pl.BlockSpec((SBR, H), lambda i: (i, 0)),
            pl.BlockSpec((SBR, 32), lambda i: (i, 0)),
        ],
        out_specs=pl.BlockSpec((SBR, H), lambda i: (i, 0)),
        out_shape=jax.ShapeDtypeStruct((T, H), jnp.float32),
    )(x, sg_w, su_w, sd_w, y0, y1, w01)


def _combine_body(ys_hbm, pos_hbm, y0_hbm, y1_hbm,
                  pos_v, bufs, sem_g, sem_w):
    wid = lax.axis_index("s") * 2 + lax.axis_index("c")
    base = wid * TPW
    pltpu.sync_copy(pos_hbm.at[pl.ds(wid * 2 * TPW, 2 * TPW)], pos_v)
    for phase, y_hbm in ((0, y0_hbm), (1, y1_hbm)):
        gth = [None] * NG
        wrt = [None] * NG

        def _gather(g, phase=phase):
            idx = pos_v[pl.ds(phase * TPW + g * 16, 16)]
            return pltpu.async_copy(ys_hbm.at[idx], bufs.at[g % 2], sem_g)

        gth[0] = _gather(0)
        for g in range(NG):
            gth[g].wait()
            if g >= 1:
                wrt[g - 1].wait()
            if g + 1 < NG:
                gth[g + 1] = _gather(g + 1)
            wrt[g] = pltpu.async_copy(
                bufs.at[g % 2], y_hbm.at[pl.ds(base + g * 16, 16)], sem_w)
        wrt[NG - 1].wait()


def _combine(ys, pos):
    mesh = plsc.VectorSubcoreMesh(core_axis_name="c", subcore_axis_name="s")
    return pl.kernel(
        _combine_body,
        mesh=mesh,
        out_type=(
            jax.ShapeDtypeStruct((T, H), jnp.float32),
            jax.ShapeDtypeStruct((T, H), jnp.float32),
        ),
        scratch_types=[
            pltpu.VMEM((2 * TPW,), jnp.int32),
            pltpu.VMEM((2, 16, H), jnp.float32),
            pltpu.SemaphoreType.DMA,
            pltpu.SemaphoreType.DMA,
        ],
    )(ys, pos)


def kernel(hidden_states, gate_w, gate_ws, up_ws, down_ws,
           shared_gate_w, shared_up_w, shared_down_w):
    metat, w01, offr, be3 = _router(hidden_states, gate_w)
    offr_flat = offr.reshape(E * 16)
    be = be3.reshape(NB)
    xs, pos = _dispatch(hidden_states, metat, offr_flat)
    ys = _mlp(be, xs, gate_ws, up_ws, down_ws)
    y0, y1 = _combine(ys, pos)
    return _shared(hidden_states, shared_gate_w, shared_up_w,
                   shared_down_w, y0, y1, w01)


# speedup vs baseline: 1.4387x; 1.0146x over previous
import functools

import jax
import jax.numpy as jnp
from jax import lax
from jax.experimental import pallas as pl
from jax.experimental.pallas import tpu as pltpu
from jax.experimental.pallas import tpu_sc as plsc

H = 2048
E = 8
F = 512
TOPK = 2
NGROUP = 4
GSZ = E // NGROUP
TOPKG = 2
SF = 1024
T = 2048

BLK = 256
NB = 24
NS = NB * BLK

TC = 8
CT = T // TC

NTILE = 32
TPW = T // NTILE
NG = TPW // 16


def _router_body(x_ref, gw_ref, metat_ref, w01_ref, offr_ref, be_ref,
                 base_ref):
    c = pl.program_id(0)

    @pl.when(c == 0)
    def _():
        base_ref[...] = jnp.zeros_like(base_ref)

    x = x_ref[...]
    gw = gw_ref[...]
    logits = lax.dot_general(
        x, gw, (((1,), (1,)), ((), ())), preferred_element_type=jnp.float32)
    m = jnp.max(logits, axis=-1, keepdims=True)
    ex = jnp.exp(logits - m)
    scores = ex / jnp.sum(ex, axis=-1, keepdims=True)

    i8 = lax.broadcasted_iota(jnp.int32, (E, E * E), 0)
    p8 = lax.broadcasted_iota(jnp.int32, (E, E * E), 1)
    gmat = jnp.where(i8 == lax.shift_right_logical(p8, 3), 1.0, 0.0)
    rmat = jnp.where(i8 == (p8 & 7), 1.0, 0.0)
    p64 = lax.broadcasted_iota(jnp.int32, (E * E, E), 0)
    e8 = lax.broadcasted_iota(jnp.int32, (E * E, E), 1)
    hmat = jnp.where((p64 & 7) == e8, 1.0, 0.0)
    pl_ = lax.broadcasted_iota(jnp.int32, (CT, E * E), 1)
    pj = lax.shift_right_logical(pl_, 3)
    pe = pl_ & 7

    def _dotn(a, b):
        return lax.dot_general(a, b, (((1,), (0,)), ((), ())),
                               preferred_element_type=jnp.float32)

    def _doth(a, b):
        return lax.dot_general(a, b, (((1,), (0,)), ((), ())),
                               preferred_element_type=jnp.float32,
                               precision=lax.Precision.HIGHEST)

    s_sw = jnp.concatenate(
        [scores[:, i ^ 1:(i ^ 1) + 1] for i in range(E)], axis=1)
    gexp = jnp.maximum(scores, s_sw)

    gb_ = _doth(gexp, gmat)
    gc_ = _doth(gexp, rmat)
    gbeat = (gb_ > gc_) | ((gb_ == gc_)
                           & (lax.shift_right_logical(pj, 1)
                              < lax.shift_right_logical(pe, 1)))
    grank = _dotn(jnp.where(gbeat, 1.0, 0.0), hmat) * 0.5
    msk = jnp.where(grank < float(TOPKG), scores, 0.0)

    eb_ = _doth(msk, gmat)
    ec_ = _doth(msk, rmat)
    ebeat = (eb_ > ec_) | ((eb_ == ec_) & (pj < pe))
    rank8 = _dotn(jnp.where(ebeat, 1.0, 0.0), hmat)

    sel0 = (rank8 == 0.0).astype(jnp.float32)
    sel1 = (rank8 == 1.0).astype(jnp.float32)

    oh = sel0 + sel1
    ii = lax.broadcasted_iota(jnp.int32, (CT, CT), 0)
    jj = lax.broadcasted_iota(jnp.int32, (CT, CT), 1)
    lexc = jnp.where(ii > jj, 1.0, 0.0)
    within = lax.dot_general(
        lexc, oh, (((1,), (0,)), ((), ())),
        preferred_element_type=jnp.float32)
    rank_te = within + base_ref[...]

    iota8c = lax.broadcasted_iota(
        jnp.int32, (E, 1), 0).astype(jnp.float32)
    ones8 = jnp.ones((E, 1), jnp.float32)

    e0 = _doth(sel0, iota8c)
    e1 = _doth(sel1, iota8c)
    w0 = _doth(sel0 * msk, ones8)
    w1 = _doth(sel1 * msk, ones8)
    r0 = _doth(sel0 * rank_te, ones8)
    r1 = _doth(sel1 * rank_te, ones8)

    zero = jnp.zeros_like(e0)
    m8 = jnp.concatenate([e0, e1, r0, r1, w0, w1, zero, zero], axis=1)
    ident = jnp.where(ii == jj, 1.0, 0.0)
    metat_ref[...] = lax.dot_general(
        m8, ident, (((0,), (0,)), ((), ())),
        preferred_element_type=jnp.float32,
        precision=lax.Precision.HIGHEST)

    w01_ref[...] = jnp.concatenate(
        [jnp.broadcast_to(w0, (CT, 16)), jnp.broadcast_to(w1, (CT, 16))],
        axis=1)

    base_ref[...] += jnp.sum(oh, axis=0, keepdims=True)

    @pl.when(c == TC - 1)
    def _():
        cnt = base_ref[...]
        padblk = jnp.floor((cnt + float(BLK - 1)) * (1.0 / BLK))
        offs = []
        run = jnp.zeros_like(padblk[:, 0:1])
        ends = []
        for e in range(E):
            offs.append(run * float(BLK))
            run = run + padblk[:, e:e + 1]
            ends.append(run)
        off2 = jnp.concatenate(offs, axis=1)
        i8a = lax.broadcasted_iota(jnp.int32, (E, E), 0)
        i8b = lax.broadcasted_iota(jnp.int32, (E, E), 1)
        ident8 = jnp.where(i8a == i8b, 1.0, 0.0)
        offcol = lax.dot_general(
            ident8, off2, (((1,), (1,)), ((), ())),
            preferred_element_type=jnp.float32,
            precision=lax.Precision.HIGHEST)
        offr_ref[...] = lax.dot_general(
            offcol, jnp.ones((1, 16), jnp.float32),
            (((1,), (0,)), ((), ())),
            preferred_element_type=jnp.float32,
            precision=lax.Precision.HIGHEST)
        bi = lax.broadcasted_iota(jnp.int32, (1, NB), 1).astype(jnp.float32)
        be = sum(jnp.where(bi >= ends[e], 1.0, 0.0) for e in range(E))
        be = jnp.minimum(be, float(E - 1))
        be_ref[...] = be.astype(jnp.int32).reshape(1, 1, NB)


def _router(x, gate_w):
    return pl.pallas_call(
        _router_body,
        grid=(TC,),
        in_specs=[
            pl.BlockSpec((CT, H), lambda c: (c, 0)),
            pl.BlockSpec((E, H), lambda c: (0, 0)),
        ],
        out_specs=(
            pl.BlockSpec((E, CT), lambda c: (0, c)),
            pl.BlockSpec((CT, 32), lambda c: (c, 0)),
            pl.BlockSpec((E, 16), lambda c: (0, 0)),
            pl.BlockSpec((1, 1, NB), lambda c: (0, 0, 0)),
        ),
        out_shape=(
            jax.ShapeDtypeStruct((E, T), jnp.float32),
            jax.ShapeDtypeStruct((T, 32), jnp.float32),
            jax.ShapeDtypeStruct((E, 16), jnp.float32),
            jax.ShapeDtypeStruct((1, 1, NB), jnp.int32),
        ),
        scratch_shapes=[pltpu.VMEM((1, E), jnp.float32)],
    )(x, gate_w)


def _dispatch_body(x_hbm, metat_hbm, offr_hbm, xs_hbm, pos_hbm,
                   mt_v, off_v, pos_v, rows_v, sem_l, sem_s):
    wid = lax.axis_index("s") * 2 + lax.axis_index("c")
    base = wid * TPW
    for r in range(4):
        pltpu.sync_copy(metat_hbm.at[r, pl.ds(base, TPW)],
                        mt_v.at[pl.ds(r * TPW, TPW)])
    pltpu.sync_copy(offr_hbm, off_v)
    posvecs = []
    for g in range(NG):
        ev0 = mt_v[pl.ds(0 * TPW + g * 16, 16)]
        ev1 = mt_v[pl.ds(1 * TPW + g * 16, 16)]
        p0 = mt_v[pl.ds(2 * TPW + g * 16, 16)]
        p1 = mt_v[pl.ds(3 * TPW + g * 16, 16)]
        for e in range(E):
            ov = off_v[pl.ds(e * 16, 16)]
            fe = float(e)
            p0 = p0 + jnp.where(ev0 == fe, ov, 0.0)
            p1 = p1 + jnp.where(ev1 == fe, ov, 0.0)
        i0 = p0.astype(jnp.int32)
        i1 = p1.astype(jnp.int32)
        pos_v[pl.ds(g * 16, 16)] = i0
        pos_v[pl.ds(TPW + g * 16, 16)] = i1
        posvecs.append((i0, i1))
    pltpu.sync_copy(pos_v, pos_hbm.at[pl.ds(wid * 2 * TPW, 2 * TPW)])
    loads = [None] * NG
    scts = [None] * NG

    def _load(g):
        return pltpu.async_copy(
            x_hbm.at[pl.ds(base + g * 16, 16)], rows_v.at[g % 2], sem_l)

    loads[0] = _load(0)
    for g in range(NG):
        loads[g].wait()
        if g >= 1:
            scts[g - 1][0].wait()
            scts[g - 1][1].wait()
        if g + 1 < NG:
            loads[g + 1] = _load(g + 1)
        i0, i1 = posvecs[g]
        scts[g] = (
            pltpu.async_copy(rows_v.at[g % 2], xs_hbm.at[i0], sem_s),
            pltpu.async_copy(rows_v.at[g % 2], xs_hbm.at[i1], sem_s),
        )
    scts[NG - 1][0].wait()
    scts[NG - 1][1].wait()


def _dispatch(x, metat, offr_flat):
    mesh = plsc.VectorSubcoreMesh(core_axis_name="c", subcore_axis_name="s")
    return pl.kernel(
        _dispatch_body,
        mesh=mesh,
        out_type=(
            jax.ShapeDtypeStruct((NS, H), jnp.float32),
            jax.ShapeDtypeStruct((2 * T,), jnp.int32),
        ),
        scratch_types=[
            pltpu.VMEM((4 * TPW,), jnp.float32),
            pltpu.VMEM((E * 16,), jnp.float32),
            pltpu.VMEM((2 * TPW,), jnp.int32),
            pltpu.VMEM((2, 16, H), jnp.float32),
            pltpu.SemaphoreType.DMA,
            pltpu.SemaphoreType.DMA,
        ],
    )(x, metat, offr_flat)


def _mlp_body(be_ref, xs_ref, wg_ref, wu_ref, wd_ref, ys_ref):
    xb = xs_ref[...]
    wg = wg_ref[0]
    wu = wu_ref[0]
    wd = wd_ref[0]
    g = lax.dot_general(xb, wg, (((1,), (1,)), ((), ())),
                        preferred_element_type=jnp.float32)
    u = lax.dot_general(xb, wu, (((1,), (1,)), ((), ())),
                        preferred_element_type=jnp.float32)
    h = g * jax.nn.sigmoid(g) * u
    ys_ref[...] = lax.dot_general(h, wd, (((1,), (1,)), ((), ())),
                                  preferred_element_type=jnp.float32)


def _mlp(be, xs, gate_ws, up_ws, down_ws):
    return pl.pallas_call(
        _mlp_body,
        grid_spec=pltpu.PrefetchScalarGridSpec(
            num_scalar_prefetch=1,
            grid=(NB,),
            in_specs=[
                pl.BlockSpec((BLK, H), lambda b, be: (b, 0)),
                pl.BlockSpec((1, F, H), lambda b, be: (be[b], 0, 0)),
                pl.BlockSpec((1, F, H), lambda b, be: (be[b], 0, 0)),
                pl.BlockSpec((1, H, F), lambda b, be: (be[b], 0, 0)),
            ],
            out_specs=pl.BlockSpec((BLK, H), lambda b, be: (b, 0)),
        ),
        out_shape=jax.ShapeDtypeStruct((NS, H), jnp.float32),
    )(be, xs, gate_ws, up_ws, down_ws)


SBT = 8
SBR = T // SBT


def _shared_body(x_ref, wg_ref, wu_ref, wd_ref, out_ref):
    xb = x_ref[...]
    wg = wg_ref[...]
    wu = wu_ref[...]
    wd = wd_ref[...]
    g = lax.dot_general(xb, wg, (((1,), (1,)), ((), ())),
                        preferred_element_type=jnp.float32)
    u = lax.dot_general(xb, wu, (((1,), (1,)), ((), ())),
                        preferred_element_type=jnp.float32)
    h = g * jax.nn.sigmoid(g) * u
    out_ref[...] = lax.dot_general(h, wd, (((1,), (1,)), ((), ())),
                                   preferred_element_type=jnp.float32)


def _shared(x, sg_w, su_w, sd_w):
    return pl.pallas_call(
        _shared_body,
        grid=(SBT,),
        in_specs=[
            pl.BlockSpec((SBR, H), lambda i: (i, 0)),
            pl.BlockSpec((SF, H), lambda i: (0, 0)),
            pl.BlockSpec((SF, H), lambda i: (0, 0)),
            pl.BlockSpec((H, SF), lambda i: (0, 0)),
        ],
        out_specs=pl.BlockSpec((SBR, H), lambda i: (i, 0)),
        out_shape=jax.ShapeDtypeStruct((T, H), jnp.float32),
    )(x, sg_w, su_w, sd_w)


def _merge_body(sh_ref, y0_ref, y1_ref, w01_ref, out_ref):
    w0 = w01_ref[:, 0:1]
    w1 = w01_ref[:, 16:17]
    out_ref[...] = sh_ref[...] + w0 * y0_ref[...] + w1 * y1_ref[...]


def _merge(shared, y0, y1, w01):
    return pl.pallas_call(
        _merge_body,
        grid=(SBT,),
        in_specs=[
            pl.BlockSpec((SBR, H), lambda i: (i, 0)),
            pl.BlockSpec((SBR, H), lambda i: (i, 0)),
            pl.BlockSpec((SBR, H), lambda i: (i, 0)),
            pl.BlockSpec((SBR, 32), lambda i: (i, 0)),
        ],
        out_specs=pl.BlockSpec((SBR, H), lambda i: (i, 0)),
        out_shape=jax.ShapeDtypeStruct((T, H), jnp.float32),
    )(shared, y0, y1, w01)


def _combine_body(ys_hbm, pos_hbm, y0_hbm, y1_hbm,
                  pos_v, bufs, sem_g, sem_w):
    wid = lax.axis_index("s") * 2 + lax.axis_index("c")
    base = wid * TPW
    pltpu.sync_copy(pos_hbm.at[pl.ds(wid * 2 * TPW, 2 * TPW)], pos_v)
    for phase, y_hbm in ((0, y0_hbm), (1, y1_hbm)):
        gth = [None] * NG
        wrt = [None] * NG

        def _gather(g, phase=phase):
            idx = pos_v[pl.ds(phase * TPW + g * 16, 16)]
            return pltpu.async_copy(ys_hbm.at[idx], bufs.at[g % 2], sem_g)

        gth[0] = _gather(0)
        for g in range(NG):
            gth[g].wait()
            if g >= 1:
                wrt[g - 1].wait()
            if g + 1 < NG:
                gth[g + 1] = _gather(g + 1)
            wrt[g] = pltpu.async_copy(
                bufs.at[g % 2], y_hbm.at[pl.ds(base + g * 16, 16)], sem_w)
        wrt[NG - 1].wait()


def _combine(ys, pos):
    mesh = plsc.VectorSubcoreMesh(core_axis_name="c", subcore_axis_name="s")
    return pl.kernel(
        _combine_body,
        mesh=mesh,
        out_type=(
            jax.ShapeDtypeStruct((T, H), jnp.float32),
            jax.ShapeDtypeStruct((T, H), jnp.float32),
        ),
        scratch_types=[
            pltpu.VMEM((2 * TPW,), jnp.int32),
            pltpu.VMEM((2, 16, H), jnp.float32),
            pltpu.SemaphoreType.DMA,
            pltpu.SemaphoreType.DMA,
        ],
    )(ys, pos)


def kernel(hidden_states, gate_w, gate_ws, up_ws, down_ws,
           shared_gate_w, shared_up_w, shared_down_w):
    metat, w01, offr, be3 = _router(hidden_states, gate_w)
    offr_flat = offr.reshape(E * 16)
    be = be3.reshape(NB)
    shared = _shared(hidden_states, shared_gate_w, shared_up_w,
                     shared_down_w)
    xs, pos = _dispatch(hidden_states, metat, offr_flat)
    ys = _mlp(be, xs, gate_ws, up_ws, down_ws)
    y0, y1 = _combine(ys, pos)
    return _merge(shared, y0, y1, w01)
